# Initial kernel scaffold; baseline (speedup 1.0000x reference)
import jax
import jax.numpy as jnp
from jax.experimental import pallas as pl


def kernel(x, edge_index, W1, b1, W2, b2):
    raise NotImplementedError("write your pallas kernel here")


# speedup vs baseline: 14.5853x; 14.5853x over previous
import functools

import jax
import jax.numpy as jnp
from jax import lax
from jax.experimental import pallas as pl
from jax.experimental.pallas import tpu as pltpu
from jax.experimental.pallas import tpu_sc as plsc

NC, NS, L = 2, 16, 16
NW = NC * NS
K_EDGE = 80
HCH = 5120


def _sc_mesh():
    return plsc.VectorSubcoreMesh(
        core_axis_name="c", subcore_axis_name="s", num_cores=NC, num_subcores=NS
    )


_SC_PARAMS = pltpu.CompilerParams(
    needs_layout_passes=False, use_tc_tiling_on_sc=False
)


def _make_hist(npad, ch, epw):
    nrow = npad // L
    nvec = epw // L
    n_pass = npad // HCH
    hrow = HCH // L

    @functools.partial(
        pl.kernel,
        out_type=jax.ShapeDtypeStruct((NC, nrow, L), jnp.float32),
        mesh=_sc_mesh(),
        scratch_types=[
            pltpu.VMEM((nvec, L), jnp.int32),
            pltpu.VMEM((L * HCH,), jnp.float32),
            pltpu.VMEM((hrow, L), jnp.float32),
            pltpu.VMEM((hrow,), jnp.int32),
            pltpu.VMEM_SHARED((nrow, L), jnp.float32),
        ],
        compiler_params=_SC_PARAMS,
    )
    def hist(dst_hbm, out_hbm, dstv, hst, red, rowidx, spacc):
        cid = lax.axis_index("c")
        sid = lax.axis_index("s")
        wid = sid * NC + cid
        zeros16 = jnp.zeros((L,), jnp.float32)
        ones16 = jnp.full((L,), 1.0, jnp.float32)
        lanes = lax.iota(jnp.int32, L)

        zr = nrow // NS

        def zred(j, c):
            red[j] = zeros16
            return c

        lax.fori_loop(0, hrow, zred, 0)
        pltpu.sync_copy(red.at[pl.ds(0, zr)], spacc.at[pl.ds(sid * zr, zr)])
        pltpu.sync_copy(dst_hbm.at[wid], dstv)
        plsc.subcore_barrier()

        for p in range(n_pass):
            lo = p * HCH

            def zh(j, c):
                hst[pl.ds(j * L, L)] = zeros16
                return c

            lax.fori_loop(0, L * HCH // L, zh, 0)

            def scat(v, c):
                d = dstv[v]
                m = (d >= lo) & (d < lo + HCH)
                col = jnp.clip(d - lo, 0, HCH - 1)
                plsc.addupdate_scatter(hst, [lanes * HCH + col], ones16, mask=m)
                return c

            lax.fori_loop(0, nvec, scat, 0)

            def rstep(j, c):
                s = hst[pl.ds(j * L, L)]
                for l in range(1, L):
                    s = s + hst[pl.ds(l * HCH + j * L, L)]
                red[j] = s
                return c

            lax.fori_loop(0, hrow, rstep, 0)

            def ridx(i, c):
                rowidx[pl.ds(i * L, L)] = lax.iota(jnp.int32, L) + (
                    p * hrow + i * L
                )
                return c

            lax.fori_loop(0, hrow // L, ridx, 0)
            pltpu.sync_copy(red, spacc.at[rowidx], add=True)

        plsc.subcore_barrier()

        @pl.when(sid == 0)
        def _():
            pltpu.sync_copy(spacc, out_hbm.at[cid])

    return hist


def _make_agg_split(npad, dh, ch):
    rows_per_tile = npad // NS
    zr = 64

    @functools.partial(
        pl.kernel,
        out_type=jax.ShapeDtypeStruct((NC, npad, dh), jnp.float32),
        mesh=_sc_mesh(),
        scratch_types=[
            pltpu.VMEM((ch, K_EDGE), jnp.int32),
            pltpu.VMEM((ch, K_EDGE), jnp.int32),
            pltpu.VMEM((K_EDGE, dh), jnp.float32),
            pltpu.VMEM((zr, dh), jnp.float32),
            pltpu.VMEM_SHARED((npad, dh), jnp.float32),
            pltpu.SemaphoreType.DMA,
        ],
        compiler_params=_SC_PARAMS,
    )
    def agg(g_hbm, src_hbm, dst_hbm, out_hbm, srcv, dstv, rows, zbuf, spacc, gsem):
        cid = lax.axis_index("c")
        sid = lax.axis_index("s")
        zeros16 = jnp.zeros((L,), jnp.float32)

        def zb(i, c):
            for j in range(dh // L):
                zbuf[i, pl.ds(j * L, L)] = zeros16
            return c

        lax.fori_loop(0, zr, zb, 0)
        base = sid * rows_per_tile
        for r in range(rows_per_tile // zr):
            pltpu.sync_copy(zbuf, spacc.at[pl.ds(base + r * zr, zr)])
        pltpu.sync_copy(src_hbm.at[sid], srcv)
        pltpu.sync_copy(dst_hbm.at[sid], dstv)
        plsc.subcore_barrier()

        gsl = g_hbm.at[cid]

        def step(i, c):
            pltpu.async_copy(gsl.at[srcv.at[i]], rows, gsem).wait()
            pltpu.sync_copy(rows, spacc.at[dstv.at[i]], add=True)
            return c

        lax.fori_loop(0, ch, step, 0)
        plsc.subcore_barrier()
        for r in range(rows_per_tile // zr):
            sl = pl.ds(base + r * zr, zr)
            pltpu.sync_copy(spacc.at[sl], out_hbm.at[cid, sl])

    return agg


def _make_agg(npad, d, ch):
    zr = 64 if d >= 64 else (npad // L)
    rows_per_tile = npad // NS

    @functools.partial(
        pl.kernel,
        out_type=jax.ShapeDtypeStruct((NC, npad, d), jnp.float32),
        mesh=_sc_mesh(),
        scratch_types=[
            pltpu.VMEM((ch, K_EDGE), jnp.int32),
            pltpu.VMEM((ch, K_EDGE), jnp.int32),
            pltpu.VMEM((K_EDGE, d), jnp.float32),
            pltpu.VMEM((zr, d), jnp.float32),
            pltpu.VMEM_SHARED((npad, d), jnp.float32),
            pltpu.SemaphoreType.DMA,
        ],
        compiler_params=_SC_PARAMS,
    )
    def agg(g_hbm, src_hbm, dst_hbm, out_hbm, srcv, dstv, rows, zbuf, spacc, gsem):
        cid = lax.axis_index("c")
        sid = lax.axis_index("s")
        wid = sid * NC + cid
        zeros16 = jnp.zeros((L,), jnp.float32)

        def zb(i, c):
            for j in range(d // L):
                zbuf[i, pl.ds(j * L, L)] = zeros16
            return c

        lax.fori_loop(0, zr, zb, 0)
        base = sid * rows_per_tile
        for r in range(rows_per_tile // zr):
            pltpu.sync_copy(zbuf, spacc.at[pl.ds(base + r * zr, zr)])
        pltpu.sync_copy(src_hbm.at[wid], srcv)
        pltpu.sync_copy(dst_hbm.at[wid], dstv)
        plsc.subcore_barrier()

        def step(i, c):
            pltpu.async_copy(g_hbm.at[srcv.at[i]], rows, gsem).wait()
            pltpu.sync_copy(rows, spacc.at[dstv.at[i]], add=True)
            return c

        lax.fori_loop(0, ch, step, 0)
        plsc.subcore_barrier()
        for r in range(rows_per_tile // zr):
            sl = pl.ds(base + r * zr, zr)
            pltpu.sync_copy(spacc.at[sl], out_hbm.at[cid, sl])

    return agg


def _transpose_col(v128):
    r = lax.broadcasted_iota(jnp.int32, (128, 128), 0)
    c = lax.broadcasted_iota(jnp.int32, (128, 128), 1)
    m = jnp.where(r == c, v128[None, :], 0.0)
    return jnp.sum(m, axis=1, keepdims=True)


def _tc_prep_body(x_ref, dp_ref, w1a_ref, w1b_ref, g_ref, dinv_ref):
    deg = dp_ref[0, 0, 0, :] + dp_ref[1, 0, 0, :] + 1.0
    dinv_col = lax.rsqrt(_transpose_col(deg))
    xb = x_ref[...]
    g_ref[0] = jnp.dot(xb, w1a_ref[...], preferred_element_type=jnp.float32) * dinv_col
    g_ref[1] = jnp.dot(xb, w1b_ref[...], preferred_element_type=jnp.float32) * dinv_col
    dinv_ref[...] = dinv_col


def _tc_mid_body(acc_ref, g_ref, dinv_ref, b1_ref, w2_ref, g2_ref):
    dinv = dinv_ref[...]
    acc = jnp.concatenate([acc_ref[0], acc_ref[1]], axis=1)
    g = jnp.concatenate([g_ref[0], g_ref[1]], axis=1)
    out1 = dinv * (acc + g) + b1_ref[...]
    h2 = jnp.maximum(out1, 0.0)
    g2_ref[...] = jnp.dot(h2, w2_ref[...], preferred_element_type=jnp.float32) * dinv


def _tc_final_body(a_ref, g2_ref, dinv_ref, b2_ref, o_ref):
    dinv = dinv_ref[...]
    acc = a_ref[0] + a_ref[1]
    o_ref[...] = dinv * (acc + g2_ref[...]) + b2_ref[...]


def kernel(x, edge_index, W1, b1, W2, b2):
    n, d_in = x.shape
    d_h = W1.shape[1]
    d_out = W2.shape[1]
    e = edge_index.shape[1]

    npad = ((n + 1279) // 1280) * 1280
    ch = -(-e // (NW * K_EDGE))
    epad = NW * ch * K_EDGE
    epw = ch * K_EDGE
    nblk = npad // 128

    src = edge_index[0]
    dst = edge_index[1]
    if epad != e:
        fill = jnp.full((epad - e,), n, dtype=jnp.int32)
        src = jnp.concatenate([src, fill])
        dst = jnp.concatenate([dst, fill])
    src3 = src.reshape(NW, ch, K_EDGE)
    dst3 = dst.reshape(NW, ch, K_EDGE)
    ch1 = NC * ch
    src16 = src.reshape(NS, ch1, K_EDGE)
    dst16 = dst.reshape(NS, ch1, K_EDGE)
    dsth = dst.reshape(NW, epw // L, L)

    xp = jnp.pad(x, ((0, npad - n), (0, 0)))
    w2p = jnp.pad(W2, ((0, 0), (0, L - d_out)))
    b2p = jnp.pad(b2, (0, L - d_out)).reshape(1, L)
    b1r = b1.reshape(1, d_h)

    dp = _make_hist(npad, ch, epw)(dsth)
    dp4 = dp.reshape(NC, nblk, 1, 128)

    dhh = d_h // NC
    w1a = W1[:, :dhh]
    w1b = W1[:, dhh:]
    gsplit, dinvc = pl.pallas_call(
        _tc_prep_body,
        grid=(nblk,),
        in_specs=[
            pl.BlockSpec((128, d_in), lambda i: (i, 0)),
            pl.BlockSpec((NC, 1, 1, 128), lambda i: (0, i, 0, 0)),
            pl.BlockSpec((d_in, dhh), lambda i: (0, 0)),
            pl.BlockSpec((d_in, dhh), lambda i: (0, 0)),
        ],
        out_specs=[
            pl.BlockSpec((NC, 128, dhh), lambda i: (0, i, 0)),
            pl.BlockSpec((128, 1), lambda i: (i, 0)),
        ],
        out_shape=[
            jax.ShapeDtypeStruct((NC, npad, dhh), jnp.float32),
            jax.ShapeDtypeStruct((npad, 1), jnp.float32),
        ],
    )(xp, dp4, w1a, w1b)

    acc1 = _make_agg_split(npad, dhh, ch1)(gsplit, src16, dst16)

    g2 = pl.pallas_call(
        _tc_mid_body,
        grid=(nblk,),
        in_specs=[
            pl.BlockSpec((NC, 128, dhh), lambda i: (0, i, 0)),
            pl.BlockSpec((NC, 128, dhh), lambda i: (0, i, 0)),
            pl.BlockSpec((128, 1), lambda i: (i, 0)),
            pl.BlockSpec((1, d_h), lambda i: (0, 0)),
            pl.BlockSpec((d_h, L), lambda i: (0, 0)),
        ],
        out_specs=pl.BlockSpec((128, L), lambda i: (i, 0)),
        out_shape=jax.ShapeDtypeStruct((npad, L), jnp.float32),
    )(acc1, gsplit, dinvc, b1r, w2p)

    acc2 = _make_agg(npad, L, ch)(g2, src3, dst3)

    out = pl.pallas_call(
        _tc_final_body,
        grid=(nblk,),
        in_specs=[
            pl.BlockSpec((NC, 128, L), lambda i: (0, i, 0)),
            pl.BlockSpec((128, L), lambda i: (i, 0)),
            pl.BlockSpec((128, 1), lambda i: (i, 0)),
            pl.BlockSpec((1, L), lambda i: (0, 0)),
        ],
        out_specs=pl.BlockSpec((128, L), lambda i: (i, 0)),
        out_shape=jax.ShapeDtypeStruct((npad, L), jnp.float32),
    )(acc2, g2, dinvc, b2p)

    return out[:n, :d_out]


# speedup vs baseline: 16.2175x; 1.1119x over previous
import functools

import jax
import jax.numpy as jnp
from jax import lax
from jax.experimental import pallas as pl
from jax.experimental.pallas import tpu as pltpu
from jax.experimental.pallas import tpu_sc as plsc

NC, NS, L = 2, 16, 16
NW = NC * NS
K_EDGE = 80
HCH = 5120


def _sc_mesh():
    return plsc.VectorSubcoreMesh(
        core_axis_name="c", subcore_axis_name="s", num_cores=NC, num_subcores=NS
    )


_SC_PARAMS = pltpu.CompilerParams(
    needs_layout_passes=False, use_tc_tiling_on_sc=False
)


def _make_hist(npad, ch, epw):
    nrow = npad // L
    nvec = epw // L
    n_pass = npad // HCH
    hrow = HCH // L

    @functools.partial(
        pl.kernel,
        out_type=jax.ShapeDtypeStruct((NC, nrow, L), jnp.float32),
        mesh=_sc_mesh(),
        scratch_types=[
            pltpu.VMEM((nvec, L), jnp.int32),
            pltpu.VMEM((L * HCH,), jnp.float32),
            pltpu.VMEM((hrow, L), jnp.float32),
            pltpu.VMEM((hrow,), jnp.int32),
            pltpu.VMEM_SHARED((nrow, L), jnp.float32),
        ],
        compiler_params=_SC_PARAMS,
    )
    def hist(dst_hbm, out_hbm, dstv, hst, red, rowidx, spacc):
        cid = lax.axis_index("c")
        sid = lax.axis_index("s")
        wid = sid * NC + cid
        zeros16 = jnp.zeros((L,), jnp.float32)
        ones16 = jnp.full((L,), 1.0, jnp.float32)
        lanes = lax.iota(jnp.int32, L)

        zr = nrow // NS

        def zred(j, c):
            red[j] = zeros16
            return c

        lax.fori_loop(0, hrow, zred, 0)
        pltpu.sync_copy(red.at[pl.ds(0, zr)], spacc.at[pl.ds(sid * zr, zr)])
        pltpu.sync_copy(dst_hbm.at[wid], dstv)
        plsc.subcore_barrier()

        for p in range(n_pass):
            lo = p * HCH

            def zh(j, c):
                hst[pl.ds(j * L, L)] = zeros16
                return c

            lax.fori_loop(0, L * HCH // L, zh, 0)

            def scat(v, c):
                d = dstv[v]
                m = (d >= lo) & (d < lo + HCH)
                col = jnp.clip(d - lo, 0, HCH - 1)
                plsc.addupdate_scatter(hst, [lanes * HCH + col], ones16, mask=m)
                return c

            lax.fori_loop(0, nvec, scat, 0)

            def rstep(j, c):
                s = hst[pl.ds(j * L, L)]
                for l in range(1, L):
                    s = s + hst[pl.ds(l * HCH + j * L, L)]
                red[j] = s
                return c

            lax.fori_loop(0, hrow, rstep, 0)

            def ridx(i, c):
                rowidx[pl.ds(i * L, L)] = lax.iota(jnp.int32, L) + (
                    p * hrow + i * L
                )
                return c

            lax.fori_loop(0, hrow // L, ridx, 0)
            pltpu.sync_copy(red, spacc.at[rowidx], add=True)

        plsc.subcore_barrier()

        @pl.when(sid == 0)
        def _():
            pltpu.sync_copy(spacc, out_hbm.at[cid])

    return hist


NBUF = 8
PREF = 4


def _make_agg(npad, d, ch, split):
    rows_per_tile = npad // NS
    zr = 64 if d >= 64 else rows_per_tile

    scratch = (
        [pltpu.VMEM((ch, K_EDGE), jnp.int32)] * 2
        + [pltpu.VMEM((K_EDGE, d), jnp.float32)] * NBUF
        + [pltpu.VMEM((zr, d), jnp.float32),
           pltpu.VMEM_SHARED((npad, d), jnp.float32)]
        + [pltpu.SemaphoreType.DMA] * (2 * NBUF)
    )

    @functools.partial(
        pl.kernel,
        out_type=jax.ShapeDtypeStruct((NC, npad, d), jnp.float32),
        mesh=_sc_mesh(),
        scratch_types=scratch,
        compiler_params=_SC_PARAMS,
    )
    def agg(g_hbm, src_hbm, dst_hbm, out_hbm, srcv, dstv, *rest):
        bufs = rest[:NBUF]
        zbuf = rest[NBUF]
        spacc = rest[NBUF + 1]
        gsems = rest[NBUF + 2 : NBUF + 2 + NBUF]
        ssems = rest[NBUF + 2 + NBUF :]
        cid = lax.axis_index("c")
        sid = lax.axis_index("s")
        w = sid if split else sid * NC + cid
        gsl = g_hbm.at[cid] if split else g_hbm
        zeros16 = jnp.zeros((L,), jnp.float32)

        def zb(i, c):
            for j in range(d // L):
                zbuf[i, pl.ds(j * L, L)] = zeros16
            return c

        lax.fori_loop(0, zr, zb, 0)
        base = sid * rows_per_tile
        for r in range(rows_per_tile // zr):
            pltpu.sync_copy(zbuf, spacc.at[pl.ds(base + r * zr, zr)])
        pltpu.sync_copy(src_hbm.at[w], srcv)
        pltpu.sync_copy(dst_hbm.at[w], dstv)
        plsc.subcore_barrier()

        for b in range(PREF):
            pltpu.async_copy(gsl.at[srcv.at[b]], bufs[b], gsems[b])

        def body(j, c):
            i0 = j * NBUF
            for b in range(NBUF):
                i = i0 + b
                pltpu.make_async_copy(gsl.at[srcv.at[i]], bufs[b], gsems[b]).wait()
                pltpu.async_copy(bufs[b], spacc.at[dstv.at[i]], ssems[b], add=True)
                ip = i + PREF
                bp = (b + PREF) % NBUF

                @pl.when(ip < ch)
                def _():
                    @pl.when(ip >= NBUF)
                    def _():
                        pltpu.make_async_copy(
                            bufs[bp], spacc.at[dstv.at[ip - NBUF]], ssems[bp]
                        ).wait()

                    pltpu.async_copy(gsl.at[srcv.at[ip]], bufs[bp], gsems[bp])

            return c

        lax.fori_loop(0, ch // NBUF, body, 0)
        for b in range(NBUF):
            pltpu.make_async_copy(
                bufs[b], spacc.at[dstv.at[ch - NBUF + b]], ssems[b]
            ).wait()
        plsc.subcore_barrier()
        for r in range(rows_per_tile // zr):
            sl = pl.ds(base + r * zr, zr)
            pltpu.sync_copy(spacc.at[sl], out_hbm.at[cid, sl])

    return agg


def _transpose_col(v128):
    r = lax.broadcasted_iota(jnp.int32, (128, 128), 0)
    c = lax.broadcasted_iota(jnp.int32, (128, 128), 1)
    m = jnp.where(r == c, v128[None, :], 0.0)
    return jnp.sum(m, axis=1, keepdims=True)


def _tc_prep_body(x_ref, dp_ref, w1a_ref, w1b_ref, g_ref, dinv_ref):
    deg = dp_ref[0, 0, 0, :] + dp_ref[1, 0, 0, :] + 1.0
    dinv_col = lax.rsqrt(_transpose_col(deg))
    xb = x_ref[...]
    g_ref[0] = jnp.dot(xb, w1a_ref[...], preferred_element_type=jnp.float32) * dinv_col
    g_ref[1] = jnp.dot(xb, w1b_ref[...], preferred_element_type=jnp.float32) * dinv_col
    dinv_ref[...] = dinv_col


def _tc_mid_body(acc_ref, g_ref, dinv_ref, b1_ref, w2_ref, g2_ref):
    dinv = dinv_ref[...]
    acc = jnp.concatenate([acc_ref[0], acc_ref[1]], axis=1)
    g = jnp.concatenate([g_ref[0], g_ref[1]], axis=1)
    out1 = dinv * (acc + g) + b1_ref[...]
    h2 = jnp.maximum(out1, 0.0)
    g2_ref[...] = jnp.dot(h2, w2_ref[...], preferred_element_type=jnp.float32) * dinv


def _tc_final_body(a_ref, g2_ref, dinv_ref, b2_ref, o_ref):
    dinv = dinv_ref[...]
    acc = a_ref[0] + a_ref[1]
    o_ref[...] = dinv * (acc + g2_ref[...]) + b2_ref[...]


def kernel(x, edge_index, W1, b1, W2, b2):
    n, d_in = x.shape
    d_h = W1.shape[1]
    d_out = W2.shape[1]
    e = edge_index.shape[1]

    npad = ((n + 1279) // 1280) * 1280
    ch = -(-e // (NW * K_EDGE))
    ch = ((ch + NBUF - 1) // NBUF) * NBUF
    epad = NW * ch * K_EDGE
    epw = epad // NW
    nblk = npad // 128

    src = edge_index[0]
    dst = edge_index[1]
    if epad != e:
        fill = jnp.full((epad - e,), n, dtype=jnp.int32)
        src = jnp.concatenate([src, fill])
        dst = jnp.concatenate([dst, fill])
    src3 = src.reshape(NW, ch, K_EDGE)
    dst3 = dst.reshape(NW, ch, K_EDGE)
    ch1 = NC * ch
    src16 = src.reshape(NS, ch1, K_EDGE)
    dst16 = dst.reshape(NS, ch1, K_EDGE)
    dsth = dst.reshape(NW, epw // L, L)

    xp = jnp.pad(x, ((0, npad - n), (0, 0)))
    w2p = jnp.pad(W2, ((0, 0), (0, L - d_out)))
    b2p = jnp.pad(b2, (0, L - d_out)).reshape(1, L)
    b1r = b1.reshape(1, d_h)

    dp = _make_hist(npad, ch, epw)(dsth)
    dp4 = dp.reshape(NC, nblk, 1, 128)

    dhh = d_h // NC
    w1a = W1[:, :dhh]
    w1b = W1[:, dhh:]
    gsplit, dinvc = pl.pallas_call(
        _tc_prep_body,
        grid=(nblk,),
        in_specs=[
            pl.BlockSpec((128, d_in), lambda i: (i, 0)),
            pl.BlockSpec((NC, 1, 1, 128), lambda i: (0, i, 0, 0)),
            pl.BlockSpec((d_in, dhh), lambda i: (0, 0)),
            pl.BlockSpec((d_in, dhh), lambda i: (0, 0)),
        ],
        out_specs=[
            pl.BlockSpec((NC, 128, dhh), lambda i: (0, i, 0)),
            pl.BlockSpec((128, 1), lambda i: (i, 0)),
        ],
        out_shape=[
            jax.ShapeDtypeStruct((NC, npad, dhh), jnp.float32),
            jax.ShapeDtypeStruct((npad, 1), jnp.float32),
        ],
    )(xp, dp4, w1a, w1b)

    acc1 = _make_agg(npad, dhh, ch1, True)(gsplit, src16, dst16)

    g2 = pl.pallas_call(
        _tc_mid_body,
        grid=(nblk,),
        in_specs=[
            pl.BlockSpec((NC, 128, dhh), lambda i: (0, i, 0)),
            pl.BlockSpec((NC, 128, dhh), lambda i: (0, i, 0)),
            pl.BlockSpec((128, 1), lambda i: (i, 0)),
            pl.BlockSpec((1, d_h), lambda i: (0, 0)),
            pl.BlockSpec((d_h, L), lambda i: (0, 0)),
        ],
        out_specs=pl.BlockSpec((128, L), lambda i: (i, 0)),
        out_shape=jax.ShapeDtypeStruct((npad, L), jnp.float32),
    )(acc1, gsplit, dinvc, b1r, w2p)

    acc2 = _make_agg(npad, L, ch, False)(g2, src3, dst3)

    out = pl.pallas_call(
        _tc_final_body,
        grid=(nblk,),
        in_specs=[
            pl.BlockSpec((NC, 128, L), lambda i: (0, i, 0)),
            pl.BlockSpec((128, L), lambda i: (i, 0)),
            pl.BlockSpec((128, 1), lambda i: (i, 0)),
            pl.BlockSpec((1, L), lambda i: (0, 0)),
        ],
        out_specs=pl.BlockSpec((128, L), lambda i: (i, 0)),
        out_shape=jax.ShapeDtypeStruct((npad, L), jnp.float32),
    )(acc2, g2, dinvc, b2p)

    return out[:n, :d_out]
